# baseline (speedup 1.0000x reference)
import functools

import jax
import jax.numpy as jnp
from jax import lax
from jax.experimental import pallas as pl
from jax.experimental.pallas import tpu as pltpu
from jax.experimental.pallas import tpu_sc as plsc

_EPS = 1e-12

try:
    _info = plsc.get_sparse_core_info()
    _NC, _NS = _info.num_cores, _info.num_subcores
except Exception:
    _NC, _NS = 2, 16
_NW = _NC * _NS


def _sc_gather(tokens, table):
    n_tok = tokens.shape[0]
    h = table.shape[1]
    per_w = n_tok // _NW
    chunk = 80
    n_chunks = per_w // chunk
    assert per_w % chunk == 0 and n_chunks % 2 == 0 and n_chunks >= 4
    mesh = plsc.VectorSubcoreMesh(core_axis_name="c", subcore_axis_name="s")

    @functools.partial(
        pl.kernel,
        out_type=jax.ShapeDtypeStruct((n_tok, h), table.dtype),
        mesh=mesh,
        scratch_types=[
            pltpu.VMEM((chunk,), jnp.int32),
            pltpu.VMEM((chunk,), jnp.int32),
            pltpu.VMEM((chunk, h), table.dtype),
            pltpu.VMEM((chunk, h), table.dtype),
            pltpu.SemaphoreType.DMA,
            pltpu.SemaphoreType.DMA,
            pltpu.SemaphoreType.DMA,
            pltpu.SemaphoreType.DMA,
        ],
    )
    def gather_kernel(tok_hbm, table_hbm, out_hbm,
                      idx0, idx1, rows0, rows1, g0, g1, s0, s1):
        wid = lax.axis_index("s") * _NC + lax.axis_index("c")
        base = wid * per_w
        bufs = ((idx0, rows0, g0, s0), (idx1, rows1, g1, s1))

        for j in range(2):
            idx, rows, g, _ = bufs[j]
            pltpu.sync_copy(tok_hbm.at[pl.ds(base + j * chunk, chunk)], idx)
            pltpu.async_copy(table_hbm.at[idx], rows, g)

        def pair_body(p, carry):
            for j in range(2):
                idx, rows, g, s = bufs[j]
                off = base + (2 * p + j) * chunk
                pltpu.make_async_copy(table_hbm.at[idx], rows, g).wait()
                pltpu.async_copy(rows, out_hbm.at[pl.ds(off, chunk)], s)
                pltpu.sync_copy(
                    tok_hbm.at[pl.ds(off + 2 * chunk, chunk)], idx)
                pltpu.make_async_copy(
                    rows, out_hbm.at[pl.ds(off, chunk)], s).wait()
                pltpu.async_copy(table_hbm.at[idx], rows, g)
            return carry

        lax.fori_loop(0, n_chunks // 2 - 1, pair_body, 0)

        for j in range(2):
            idx, rows, g, _ = bufs[j]
            off = base + (n_chunks - 2 + j) * chunk
            pltpu.make_async_copy(table_hbm.at[idx], rows, g).wait()
            pltpu.sync_copy(rows, out_hbm.at[pl.ds(off, chunk)])

    return gather_kernel(tokens, table)


def _tc_fuse(word_rows, spatial_t, tmask_t, pos3, type_emb, proj_w,
             proj_b, ln_gamma, ln_beta, bb, block_off, prev_out):
    l_all, bpart, h = word_rows.shape
    b = spatial_t.shape[1]
    lq = spatial_t.shape[0]
    s = spatial_t.shape[2]
    grid = (bpart // bb,)

    def _ln(x, g, be):
        mean = jnp.mean(x, axis=-1, keepdims=True)
        cent = x - mean
        var = jnp.mean(cent * cent, axis=-1, keepdims=True)
        return cent * lax.rsqrt(var + _EPS) * g + be

    def body(wr_ref, sp_ref, tm_ref, pos_ref, te_ref, w_ref, pb_ref, g_ref,
             be_ref, *rest):
        out_ref = rest[-1]
        g = g_ref[...].reshape(1, 1, h)
        be = be_ref[...].reshape(1, 1, h)

        te0 = te_ref[0:1, :].reshape(1, 1, h)
        te_d = (te_ref[1:2, :] - te_ref[0:1, :]).reshape(1, 1, h)
        q = wr_ref[:lq] + pos_ref[...] + te0 + tm_ref[...] * te_d
        out_ref[:lq] = _ln(q, g, be)

        sp2 = sp_ref[...].reshape(lq * bb, s)
        proj = jnp.dot(sp2, w_ref[...].astype(jnp.bfloat16),
                       preferred_element_type=jnp.float32)
        i_emb = wr_ref[lq:] + proj.reshape(lq, bb, h) \
            + pb_ref[...].reshape(1, 1, h)
        out_ref[lq:] = _ln(i_emb, g, be)

    in_specs = [
        pl.BlockSpec((l_all, bb, h), lambda i: (0, i, 0)),
        pl.BlockSpec((lq, bb, s), lambda i, o=block_off: (0, i + o, 0)),
        pl.BlockSpec((lq, bb, 1), lambda i, o=block_off: (0, i + o, 0)),
        pl.BlockSpec((lq, 1, h), lambda i: (0, 0, 0)),
        pl.BlockSpec((2, h), lambda i: (0, 0)),
        pl.BlockSpec((s, h), lambda i: (0, 0)),
        pl.BlockSpec((1, h), lambda i: (0, 0)),
        pl.BlockSpec((1, h), lambda i: (0, 0)),
        pl.BlockSpec((1, h), lambda i: (0, 0)),
    ]
    args = [word_rows, spatial_t, tmask_t, pos3, type_emb, proj_w,
            proj_b, ln_gamma, ln_beta]
    aliases = {}
    if prev_out is not None:
        in_specs.append(pl.BlockSpec(memory_space=pl.ANY))
        args.append(prev_out)
        aliases = {9: 0}
    return pl.pallas_call(
        body,
        grid=grid,
        in_specs=in_specs,
        out_specs=pl.BlockSpec((l_all, bb, h),
                               lambda i, o=block_off: (0, i + o, 0)),
        out_shape=jax.ShapeDtypeStruct((l_all, b, h), jnp.float32),
        input_output_aliases=aliases,
        compiler_params=pltpu.CompilerParams(
            dimension_semantics=("parallel",),
        ),
    )(*args)


def kernel(question_tokens, image_tokens, spatial_embeddings, token_type_ids,
           word_emb, pos_emb, type_emb, proj_W, proj_b, ln_gamma, ln_beta):
    b, lq = question_tokens.shape
    li = image_tokens.shape[1]
    v, h = word_emb.shape
    s = spatial_embeddings.shape[2]
    l_all = lq + li

    tok_t = jnp.concatenate(
        [jnp.clip(question_tokens, 0, v - 1),
         jnp.clip(image_tokens, 0, v - 1)], axis=1).T
    sp_t = spatial_embeddings.astype(jnp.bfloat16).transpose(1, 0, 2)
    tmask_t = jnp.clip(token_type_ids, 0, 1).astype(jnp.float32) \
        .T.reshape(lq, b, 1)
    pos3 = pos_emb[:lq].reshape(lq, 1, h)

    n_parts, bb = 4, 16
    bpart = b // n_parts
    stages = [
        _sc_gather(
            lax.slice_in_dim(tok_t, k * bpart, (k + 1) * bpart, axis=1)
            .reshape(l_all * bpart),
            word_emb).reshape(l_all, bpart, h)
        for k in range(n_parts)
    ]
    out = None
    for k in range(n_parts):
        out = _tc_fuse(stages[k], sp_t, tmask_t, pos3, type_emb, proj_W,
                       proj_b.reshape(1, h), ln_gamma.reshape(1, h),
                       ln_beta.reshape(1, h), bb=bb,
                       block_off=k * (bpart // bb), prev_out=out)
    return out.transpose(1, 0, 2)


# speedup vs baseline: 1.0139x; 1.0139x over previous
import functools

import jax
import jax.numpy as jnp
from jax import lax
from jax.experimental import pallas as pl
from jax.experimental.pallas import tpu as pltpu
from jax.experimental.pallas import tpu_sc as plsc

_EPS = 1e-12

try:
    _info = plsc.get_sparse_core_info()
    _NC, _NS = _info.num_cores, _info.num_subcores
except Exception:
    _NC, _NS = 2, 16
_NW = _NC * _NS


def _sc_gather(tokens, table):
    n_tok = tokens.shape[0]
    h = table.shape[1]
    per_w = n_tok // _NW
    chunk = 80
    n_chunks = per_w // chunk
    assert per_w % chunk == 0 and n_chunks % 2 == 0 and n_chunks >= 4
    mesh = plsc.VectorSubcoreMesh(core_axis_name="c", subcore_axis_name="s")

    @functools.partial(
        pl.kernel,
        out_type=jax.ShapeDtypeStruct((n_tok, h), table.dtype),
        mesh=mesh,
        scratch_types=[
            pltpu.VMEM((chunk,), jnp.int32),
            pltpu.VMEM((chunk,), jnp.int32),
            pltpu.VMEM((chunk, h), table.dtype),
            pltpu.VMEM((chunk, h), table.dtype),
            pltpu.SemaphoreType.DMA,
            pltpu.SemaphoreType.DMA,
            pltpu.SemaphoreType.DMA,
            pltpu.SemaphoreType.DMA,
        ],
    )
    def gather_kernel(tok_hbm, table_hbm, out_hbm,
                      idx0, idx1, rows0, rows1, g0, g1, s0, s1):
        wid = lax.axis_index("s") * _NC + lax.axis_index("c")
        base = wid * per_w
        bufs = ((idx0, rows0, g0, s0), (idx1, rows1, g1, s1))

        for j in range(2):
            idx, rows, g, _ = bufs[j]
            pltpu.sync_copy(tok_hbm.at[pl.ds(base + j * chunk, chunk)], idx)
            pltpu.async_copy(table_hbm.at[idx], rows, g)

        def pair_body(p, carry):
            for j in range(2):
                idx, rows, g, s = bufs[j]
                off = base + (2 * p + j) * chunk
                pltpu.make_async_copy(table_hbm.at[idx], rows, g).wait()
                pltpu.async_copy(rows, out_hbm.at[pl.ds(off, chunk)], s)
                pltpu.sync_copy(
                    tok_hbm.at[pl.ds(off + 2 * chunk, chunk)], idx)
                pltpu.make_async_copy(
                    rows, out_hbm.at[pl.ds(off, chunk)], s).wait()
                pltpu.async_copy(table_hbm.at[idx], rows, g)
            return carry

        lax.fori_loop(0, n_chunks // 2 - 1, pair_body, 0)

        for j in range(2):
            idx, rows, g, _ = bufs[j]
            off = base + (n_chunks - 2 + j) * chunk
            pltpu.make_async_copy(table_hbm.at[idx], rows, g).wait()
            pltpu.sync_copy(rows, out_hbm.at[pl.ds(off, chunk)])

    return gather_kernel(tokens, table)


def _tc_fuse(word_rows, spatial_t, tmask_t, pos3, type_emb, proj_w,
             proj_b, ln_gamma, ln_beta, bb, block_off, prev_out):
    l_all, bpart, h = word_rows.shape
    b = spatial_t.shape[1]
    lq = spatial_t.shape[0]
    s = spatial_t.shape[2]
    grid = (bpart // bb,)

    def _ln(x, g, be):
        mean = jnp.mean(x, axis=-1, keepdims=True)
        cent = x - mean
        var = jnp.mean(cent * cent, axis=-1, keepdims=True)
        return cent * lax.rsqrt(var + _EPS) * g + be

    def body(wr_ref, sp_ref, tm_ref, pos_ref, te_ref, w_ref, pb_ref, g_ref,
             be_ref, *rest):
        out_ref = rest[-1]
        g = g_ref[...].reshape(1, 1, h)
        be = be_ref[...].reshape(1, 1, h)

        te0 = te_ref[0:1, :].reshape(1, 1, h)
        te_d = (te_ref[1:2, :] - te_ref[0:1, :]).reshape(1, 1, h)
        q = wr_ref[:lq] + pos_ref[...] + te0 + tm_ref[...] * te_d
        out_ref[:lq] = _ln(q, g, be)

        sp2 = sp_ref[...].reshape(lq * bb, s)
        proj = jnp.dot(sp2, w_ref[...].astype(jnp.bfloat16),
                       preferred_element_type=jnp.float32)
        i_emb = wr_ref[lq:] + proj.reshape(lq, bb, h) \
            + pb_ref[...].reshape(1, 1, h)
        out_ref[lq:] = _ln(i_emb, g, be)

    in_specs = [
        pl.BlockSpec((l_all, bb, h), lambda i: (0, i, 0)),
        pl.BlockSpec((lq, bb, s), lambda i, o=block_off: (0, i + o, 0)),
        pl.BlockSpec((lq, bb, 1), lambda i, o=block_off: (0, i + o, 0)),
        pl.BlockSpec((lq, 1, h), lambda i: (0, 0, 0)),
        pl.BlockSpec((2, h), lambda i: (0, 0)),
        pl.BlockSpec((s, h), lambda i: (0, 0)),
        pl.BlockSpec((1, h), lambda i: (0, 0)),
        pl.BlockSpec((1, h), lambda i: (0, 0)),
        pl.BlockSpec((1, h), lambda i: (0, 0)),
    ]
    args = [word_rows, spatial_t, tmask_t, pos3, type_emb, proj_w,
            proj_b, ln_gamma, ln_beta]
    aliases = {}
    if prev_out is not None:
        in_specs.append(pl.BlockSpec(memory_space=pl.ANY))
        args.append(prev_out)
        aliases = {9: 0}
    return pl.pallas_call(
        body,
        grid=grid,
        in_specs=in_specs,
        out_specs=pl.BlockSpec((l_all, bb, h),
                               lambda i, o=block_off: (0, i + o, 0)),
        out_shape=jax.ShapeDtypeStruct((l_all, b, h), jnp.float32),
        input_output_aliases=aliases,
        compiler_params=pltpu.CompilerParams(
            dimension_semantics=("parallel",),
        ),
    )(*args)


def kernel(question_tokens, image_tokens, spatial_embeddings, token_type_ids,
           word_emb, pos_emb, type_emb, proj_W, proj_b, ln_gamma, ln_beta):
    b, lq = question_tokens.shape
    li = image_tokens.shape[1]
    v, h = word_emb.shape
    s = spatial_embeddings.shape[2]
    l_all = lq + li

    tok_t = jnp.concatenate(
        [jnp.clip(question_tokens, 0, v - 1),
         jnp.clip(image_tokens, 0, v - 1)], axis=1).T
    sp_t = spatial_embeddings.astype(jnp.bfloat16).transpose(1, 0, 2)
    tmask_t = jnp.clip(token_type_ids, 0, 1).astype(jnp.float32) \
        .T.reshape(lq, b, 1)
    pos3 = pos_emb[:lq].reshape(lq, 1, h)

    n_parts, bb = 2, 16
    bpart = b // n_parts
    stages = [
        _sc_gather(
            lax.slice_in_dim(tok_t, k * bpart, (k + 1) * bpart, axis=1)
            .reshape(l_all * bpart),
            word_emb).reshape(l_all, bpart, h)
        for k in range(n_parts)
    ]
    out = None
    for k in range(n_parts):
        out = _tc_fuse(stages[k], sp_t, tmask_t, pos3, type_emb, proj_W,
                       proj_b.reshape(1, h), ln_gamma.reshape(1, h),
                       ln_beta.reshape(1, h), bb=bb,
                       block_off=k * (bpart // bb), prev_out=out)
    return out.transpose(1, 0, 2)
